# baseline (speedup 1.0000x reference)
import functools

import jax
import jax.numpy as jnp
from jax import lax
from jax.experimental import pallas as pl
from jax.experimental.pallas import tpu as pltpu
from jax.experimental.pallas import tpu_sc as plsc

_N = 10000
_E = 320000
_NG = 64
_NC = 2
_NS = 16
_NW = _NC * _NS
_C = 128
_EPAD = 327680
_NCHROWS = _EPAD // _C
_NCH1 = _NCHROWS // _NW
_NCH2 = _NCHROWS // _NS
_NPAD = 10240
_RPT = _NPAD // _NS
_SLAB = 128
_NB = 4

_mesh = plsc.VectorSubcoreMesh(core_axis_name="c", subcore_axis_name="s")
_sc_params = pltpu.CompilerParams(use_tc_tiling_on_sc=False)


def _zero_slab(slab, d):
  def zrow(r, carry):
    def zcol(j, c2):
      slab[r, pl.ds(j * 16, 16)] = jnp.zeros((16,), jnp.float32)
      return c2
    return lax.fori_loop(0, d // 16, zcol, carry)
  lax.fori_loop(0, _SLAB, zrow, 0)


def _zero_acc_slab(slab, acc, s, d):
  _zero_slab(slab, d)
  for i in range(_RPT // _SLAB):
    pltpu.sync_copy(slab, acc.at[pl.ds(s * _RPT + i * _SLAB, _SLAB)])


def _dump_acc_slab(slab, acc, out_hbm, c, s):
  for i in range(_RPT // _SLAB):
    pltpu.sync_copy(acc.at[pl.ds(s * _RPT + i * _SLAB, _SLAB)], slab)
    pltpu.sync_copy(slab, out_hbm.at[pl.ds(c * _NPAD + s * _RPT + i * _SLAB,
                                           _SLAB)])


def _gs_pipeline(y_hbm, sidx_v, didx_v, rows, acc, gsem, ssem, nch):
  nb = len(rows)
  half = nb // 2
  banks = [rows[:half], rows[half:]]
  ng = nch // half

  def fire_g(g, bank):
    for b in range(half):
      for h in range(2):
        pltpu.async_copy(y_hbm.at[sidx_v.at[(g * half + b) * 2 + h]],
                         bank[b].at[pl.ds(h * 64, 64)], gsem)

  def wait_g(g, bank):
    for b in range(half):
      for h in range(2):
        pltpu.make_async_copy(y_hbm.at[sidx_v.at[(g * half + b) * 2 + h]],
                              bank[b].at[pl.ds(h * 64, 64)], gsem).wait()

  def fire_s(g, bank):
    for b in range(half):
      pltpu.async_copy(bank[b], acc.at[didx_v.at[g * half + b]], ssem,
                       add=True)

  def wait_s(g, bank):
    for b in range(half):
      pltpu.make_async_copy(bank[b], acc.at[didx_v.at[g * half + b]],
                            ssem).wait()

  fire_g(0, banks[0])
  fire_g(1, banks[1])

  def body(p, carry):
    ga = 2 * p
    gb = 2 * p + 1
    wait_g(ga, banks[0])
    fire_s(ga, banks[0])
    wait_g(gb, banks[1])
    fire_s(gb, banks[1])
    wait_s(ga, banks[0])

    @pl.when(ga + 2 < ng)
    def _():
      fire_g(ga + 2, banks[0])

    wait_s(gb, banks[1])

    @pl.when(gb + 2 < ng)
    def _():
      fire_g(gb + 2, banks[1])

    return carry

  lax.fori_loop(0, ng // 2, body, 0)


def _make_scatter_half(dh):
  @functools.partial(
      pl.kernel,
      mesh=_mesh,
      out_type=jax.ShapeDtypeStruct((_NC * _NPAD, dh), jnp.float32),
      scratch_types=(
          [pltpu.VMEM((_NCH2 * 2, 64), jnp.int32),
           pltpu.VMEM((_NCH2, _C), jnp.int32)]
          + [pltpu.VMEM((_C, dh), jnp.float32)] * _NB
          + [pltpu.VMEM((_SLAB, dh), jnp.float32),
             pltpu.VMEM_SHARED((_NPAD, dh), jnp.float32),
             pltpu.SemaphoreType.DMA,
             pltpu.SemaphoreType.DMA]
      ),
      compiler_params=_sc_params,
  )
  def k(ya_hbm, yb_hbm, src_hbm, dst_hbm, out_hbm, sidx_v, didx_v,
        r0, r1, r2, r3, slab, acc, gsem, ssem):
    c = lax.axis_index("c")
    s = lax.axis_index("s")

    pltpu.sync_copy(src_hbm.at[pl.ds(s * _NCH2 * 2, _NCH2 * 2)], sidx_v)
    pltpu.sync_copy(dst_hbm.at[pl.ds(s * _NCH2, _NCH2)], didx_v)
    _zero_acc_slab(slab, acc, s, dh)
    plsc.subcore_barrier()

    rows = [r0, r1, r2, r3]

    @pl.when(c == 0)
    def _():
      _gs_pipeline(ya_hbm, sidx_v, didx_v, rows, acc, gsem, ssem, _NCH2)

    @pl.when(c == 1)
    def _():
      _gs_pipeline(yb_hbm, sidx_v, didx_v, rows, acc, gsem, ssem, _NCH2)

    plsc.subcore_barrier()
    _dump_acc_slab(slab, acc, out_hbm, c, s)

  return k


_scat2 = _make_scatter_half(64)


@functools.partial(
    pl.kernel,
    mesh=_mesh,
    out_type=jax.ShapeDtypeStruct((_NC * _NPAD, 64), jnp.float32),
    scratch_types=(
        [pltpu.VMEM((_NCH1 * 2, 64), jnp.int32),
         pltpu.VMEM((_NCH1, _C), jnp.int32)]
        + [pltpu.VMEM((_C, 64), jnp.float32)] * _NB
        + [pltpu.VMEM((_SLAB, 64), jnp.float32),
           pltpu.VMEM_SHARED((_NPAD, 64), jnp.float32),
           pltpu.SemaphoreType.DMA,
           pltpu.SemaphoreType.DMA]
    ),
    compiler_params=_sc_params,
)
def _scat1es(y_hbm, src_hbm, dst_hbm, out_hbm, sidx_v, didx_v,
             r0, r1, r2, r3, slab, acc, gsem, ssem):
  c = lax.axis_index("c")
  s = lax.axis_index("s")
  wid = s * _NC + c

  pltpu.sync_copy(src_hbm.at[pl.ds(wid * _NCH1 * 2, _NCH1 * 2)], sidx_v)
  pltpu.sync_copy(dst_hbm.at[pl.ds(wid * _NCH1, _NCH1)], didx_v)
  _zero_acc_slab(slab, acc, s, 64)
  plsc.subcore_barrier()

  _gs_pipeline(y_hbm, sidx_v, didx_v, [r0, r1, r2, r3], acc, gsem, ssem,
               _NCH1)
  plsc.subcore_barrier()
  _dump_acc_slab(slab, acc, out_hbm, c, s)


@functools.partial(
    pl.kernel,
    mesh=_mesh,
    out_type=jax.ShapeDtypeStruct((_NC * _NPAD, 16), jnp.float32),
    scratch_types=[
        pltpu.VMEM((_NCH1, _C), jnp.int32),
        pltpu.VMEM((_C, 16), jnp.float32),
        pltpu.VMEM((_SLAB, 16), jnp.float32),
        pltpu.VMEM_SHARED((_NPAD, 16), jnp.float32),
        pltpu.SemaphoreType.DMA,
    ],
    compiler_params=_sc_params,
)
def _hist(dst_hbm, out_hbm, didx_v, ones_v, slab, acc, ssem):
  c = lax.axis_index("c")
  s = lax.axis_index("s")
  wid = s * _NC + c

  def orow(r, carry):
    ones_v[r, pl.ds(0, 16)] = jnp.ones((16,), jnp.float32)
    return carry
  lax.fori_loop(0, _C, orow, 0)

  pltpu.sync_copy(dst_hbm.at[pl.ds(wid * _NCH1, _NCH1)], didx_v)
  _zero_acc_slab(slab, acc, s, 16)
  plsc.subcore_barrier()

  nb = 8

  def group(t, carry):
    j0 = t * nb
    for b in range(nb):
      pltpu.async_copy(ones_v, acc.at[didx_v.at[j0 + b]], ssem, add=True)
    for b in range(nb):
      pltpu.make_async_copy(ones_v, acc.at[didx_v.at[j0 + b]], ssem).wait()
    return carry

  lax.fori_loop(0, _NCH1 // nb, group, 0)
  plsc.subcore_barrier()
  _dump_acc_slab(slab, acc, out_hbm, c, s)


_BM = 1000
_NBLK = _N // _BM


def _dis_of(ha, hb):
  deg = ha[:, 0:1] + hb[:, 0:1] + 1.0
  return lax.rsqrt(deg)


def _k1_body(x_ref, w_ref, ha_ref, hb_ref, o_ref):
  dis = _dis_of(ha_ref[...], hb_ref[...])
  xw = jnp.dot(x_ref[...], w_ref[...], preferred_element_type=jnp.float32)
  o_ref[...] = xw * dis


def _k2_body(sa, sb, y1r, ha, hb, w2, b1r, oa, ob):
  dis = _dis_of(ha[...], hb[...])
  h = jnp.maximum((sa[...] + sb[...] + y1r[...]) * dis + b1r[...], 0.0)
  y2 = jnp.dot(h, w2[...], preferred_element_type=jnp.float32) * dis
  oa[...] = y2[:, :64]
  ob[...] = y2[:, 64:]


def _k3_body(sa, sb, y2a, y2b, ha, hb, b2r, batchr, fcw, fcb, o, sums, cnts):
  g = pl.program_id(0)

  @pl.when(g == 0)
  def _():
    sums[...] = jnp.zeros_like(sums)
    cnts[...] = jnp.zeros_like(cnts)

  dis = _dis_of(ha[...], hb[...])
  h_lo = jnp.maximum((sa[...] + y2a[...]) * dis + b2r[:, :64], 0.0)
  h_hi = jnp.maximum((sb[...] + y2b[...]) * dis + b2r[:, 64:], 0.0)
  h = jnp.concatenate([h_lo, h_hi], axis=1)
  gid = lax.broadcasted_iota(jnp.int32, (_BM, _NG), 1)
  p = (batchr[...] == gid).astype(jnp.float32)
  dn = (((0,), (0,)), ((), ()))
  sums[...] += lax.dot_general(p, h, dn, preferred_element_type=jnp.float32)
  cnts[...] += lax.dot_general(p, jnp.ones_like(h), dn,
                               preferred_element_type=jnp.float32)

  @pl.when(g == _NBLK - 1)
  def _():
    pooled = sums[...] / jnp.maximum(cnts[...], 1.0)
    o[...] = jnp.dot(pooled, fcw[...],
                     preferred_element_type=jnp.float32) + fcb[...]


def _row_spec(d):
  return pl.BlockSpec((_BM, d), lambda i: (i, 0))


def _full_spec(shape):
  return pl.BlockSpec(shape, lambda i: tuple(0 for _ in shape))


def kernel(x, edge_index, batch, W1, b1, W2, b2, fc_W, fc_b):
  src = edge_index[0].astype(jnp.int32)
  dst = edge_index[1].astype(jnp.int32)
  npad = _EPAD - _E
  src2d = jnp.concatenate([src, jnp.zeros((npad,), jnp.int32)]).reshape(
      _NCHROWS * 2, 64)
  dst2d = jnp.concatenate([dst, jnp.full((npad,), _N, jnp.int32)]).reshape(
      _NCHROWS, _C)

  hist = _hist(dst2d)
  ha, hb = hist[:_N], hist[_NPAD:_NPAD + _N]

  y1 = pl.pallas_call(
      _k1_body,
      grid=(_NBLK,),
      in_specs=[_row_spec(128), _full_spec((128, 64)),
                _row_spec(16), _row_spec(16)],
      out_specs=_row_spec(64),
      out_shape=jax.ShapeDtypeStruct((_N, 64), jnp.float32),
  )(x, W1, ha, hb)

  s1 = _scat1es(y1, src2d, dst2d)

  y2a, y2b = pl.pallas_call(
      _k2_body,
      grid=(_NBLK,),
      in_specs=[_row_spec(64), _row_spec(64), _row_spec(64),
                _row_spec(16), _row_spec(16),
                _full_spec((64, 128)), _full_spec((1, 64))],
      out_specs=[_row_spec(64), _row_spec(64)],
      out_shape=[jax.ShapeDtypeStruct((_N, 64), jnp.float32),
                 jax.ShapeDtypeStruct((_N, 64), jnp.float32)],
  )(s1[:_N], s1[_NPAD:_NPAD + _N], y1, ha, hb, W2, b1.reshape(1, 64))

  s2 = _scat2(y2a, y2b, src2d, dst2d)

  out = pl.pallas_call(
      _k3_body,
      grid=(_NBLK,),
      in_specs=[_row_spec(64), _row_spec(64), _row_spec(64), _row_spec(64),
                _row_spec(16), _row_spec(16),
                _full_spec((1, 128)), pl.BlockSpec((_BM, 1), lambda i: (i, 0)),
                _full_spec((128, 2)), _full_spec((1, 2))],
      out_specs=_full_spec((_NG, 2)),
      out_shape=jax.ShapeDtypeStruct((_NG, 2), jnp.float32),
      scratch_shapes=[pltpu.VMEM((_NG, 128), jnp.float32),
                      pltpu.VMEM((_NG, 128), jnp.float32)],
  )(s2[:_N], s2[_NPAD:_NPAD + _N], y2a, y2b, ha, hb, b2.reshape(1, 128),
    batch.astype(jnp.int32).reshape(_N, 1), fc_W, fc_b.reshape(1, 2))

  return out


# speedup vs baseline: 1.2330x; 1.2330x over previous
import functools

import jax
import jax.numpy as jnp
from jax import lax
from jax.experimental import pallas as pl
from jax.experimental.pallas import tpu as pltpu
from jax.experimental.pallas import tpu_sc as plsc

_N = 10000
_E = 320000
_NG = 64
_NC = 2
_NS = 16
_NW = _NC * _NS
_C = 128
_EPAD = 327680
_NCHROWS = _EPAD // _C
_NCH1 = _NCHROWS // _NW
_NCH2 = _NCHROWS // _NS
_NPAD = 10240
_RPT = _NPAD // _NS
_SLAB = 128
_NB = 4

_mesh = plsc.VectorSubcoreMesh(core_axis_name="c", subcore_axis_name="s")
_sc_params = pltpu.CompilerParams(use_tc_tiling_on_sc=False)


def _zero_slab(slab, d):
  def zrow(r, carry):
    def zcol(j, c2):
      slab[r, pl.ds(j * 16, 16)] = jnp.zeros((16,), jnp.float32)
      return c2
    return lax.fori_loop(0, d // 16, zcol, carry)
  lax.fori_loop(0, _SLAB, zrow, 0)


def _zero_acc_slab(slab, acc, s, d):
  _zero_slab(slab, d)
  for i in range(_RPT // _SLAB):
    pltpu.sync_copy(slab, acc.at[pl.ds(s * _RPT + i * _SLAB, _SLAB)])


def _dump_acc_slab(slab, acc, out_hbm, c, s):
  for i in range(_RPT // _SLAB):
    pltpu.sync_copy(acc.at[pl.ds(s * _RPT + i * _SLAB, _SLAB)], slab)
    pltpu.sync_copy(slab, out_hbm.at[c, pl.ds(s * _RPT + i * _SLAB, _SLAB)])


def _gs_pipeline(y_hbm, sidx_v, didx_v, rows, acc, gsem, ssem, nch):
  nb = len(rows)
  half = nb // 2
  banks = [rows[:half], rows[half:]]
  ng = nch // half

  def fire_g(g, bank):
    for b in range(half):
      for h in range(2):
        pltpu.async_copy(y_hbm.at[sidx_v.at[(g * half + b) * 2 + h]],
                         bank[b].at[pl.ds(h * 64, 64)], gsem)

  def wait_g(g, bank):
    for b in range(half):
      for h in range(2):
        pltpu.make_async_copy(y_hbm.at[sidx_v.at[(g * half + b) * 2 + h]],
                              bank[b].at[pl.ds(h * 64, 64)], gsem).wait()

  def fire_s(g, bank):
    for b in range(half):
      pltpu.async_copy(bank[b], acc.at[didx_v.at[g * half + b]], ssem,
                       add=True)

  def wait_s(g, bank):
    for b in range(half):
      pltpu.make_async_copy(bank[b], acc.at[didx_v.at[g * half + b]],
                            ssem).wait()

  fire_g(0, banks[0])
  fire_g(1, banks[1])

  def body(p, carry):
    ga = 2 * p
    gb = 2 * p + 1
    wait_g(ga, banks[0])
    fire_s(ga, banks[0])
    wait_g(gb, banks[1])
    fire_s(gb, banks[1])
    wait_s(ga, banks[0])

    @pl.when(ga + 2 < ng)
    def _():
      fire_g(ga + 2, banks[0])

    wait_s(gb, banks[1])

    @pl.when(gb + 2 < ng)
    def _():
      fire_g(gb + 2, banks[1])

    return carry

  lax.fori_loop(0, ng // 2, body, 0)


def _make_scatter_half(dh):
  @functools.partial(
      pl.kernel,
      mesh=_mesh,
      out_type=jax.ShapeDtypeStruct((_NC, _NPAD, dh), jnp.float32),
      scratch_types=(
          [pltpu.VMEM((_NCH2 * 2, 64), jnp.int32),
           pltpu.VMEM((_NCH2, _C), jnp.int32)]
          + [pltpu.VMEM((_C, dh), jnp.float32)] * _NB
          + [pltpu.VMEM((_SLAB, dh), jnp.float32),
             pltpu.VMEM_SHARED((_NPAD, dh), jnp.float32),
             pltpu.SemaphoreType.DMA,
             pltpu.SemaphoreType.DMA]
      ),
      compiler_params=_sc_params,
  )
  def k(ya_hbm, yb_hbm, src_hbm, dst_hbm, out_hbm, sidx_v, didx_v,
        r0, r1, r2, r3, slab, acc, gsem, ssem):
    c = lax.axis_index("c")
    s = lax.axis_index("s")

    pltpu.sync_copy(src_hbm.at[pl.ds(s * _NCH2 * 2, _NCH2 * 2)], sidx_v)
    pltpu.sync_copy(dst_hbm.at[pl.ds(s * _NCH2, _NCH2)], didx_v)
    _zero_acc_slab(slab, acc, s, dh)
    plsc.subcore_barrier()

    rows = [r0, r1, r2, r3]

    @pl.when(c == 0)
    def _():
      _gs_pipeline(ya_hbm, sidx_v, didx_v, rows, acc, gsem, ssem, _NCH2)

    @pl.when(c == 1)
    def _():
      _gs_pipeline(yb_hbm, sidx_v, didx_v, rows, acc, gsem, ssem, _NCH2)

    plsc.subcore_barrier()
    _dump_acc_slab(slab, acc, out_hbm, c, s)

  return k


_scat1 = _make_scatter_half(32)
_scat2 = _make_scatter_half(64)


@functools.partial(
    pl.kernel,
    mesh=_mesh,
    out_type=jax.ShapeDtypeStruct((_NC, _NPAD, 16), jnp.float32),
    scratch_types=[
        pltpu.VMEM((_NCH1, _C), jnp.int32),
        pltpu.VMEM((_C, 16), jnp.float32),
        pltpu.VMEM((_SLAB, 16), jnp.float32),
        pltpu.VMEM_SHARED((_NPAD, 16), jnp.float32),
        pltpu.SemaphoreType.DMA,
    ],
    compiler_params=_sc_params,
)
def _hist(dst_hbm, out_hbm, didx_v, ones_v, slab, acc, ssem):
  c = lax.axis_index("c")
  s = lax.axis_index("s")
  wid = s * _NC + c

  def orow(r, carry):
    ones_v[r, pl.ds(0, 16)] = jnp.ones((16,), jnp.float32)
    return carry
  lax.fori_loop(0, _C, orow, 0)

  pltpu.sync_copy(dst_hbm.at[pl.ds(wid * _NCH1, _NCH1)], didx_v)
  _zero_acc_slab(slab, acc, s, 16)
  plsc.subcore_barrier()

  nb = 8

  def group(t, carry):
    j0 = t * nb
    for b in range(nb):
      pltpu.async_copy(ones_v, acc.at[didx_v.at[j0 + b]], ssem, add=True)
    for b in range(nb):
      pltpu.make_async_copy(ones_v, acc.at[didx_v.at[j0 + b]], ssem).wait()
    return carry

  lax.fori_loop(0, _NCH1 // nb, group, 0)
  plsc.subcore_barrier()
  _dump_acc_slab(slab, acc, out_hbm, c, s)


_BM = 1000
_NBLK = _N // _BM


def _dis_of(ha, hb):
  deg = ha[0, :, 0:1] + hb[0, :, 0:1] + 1.0
  return lax.rsqrt(deg)


def _k1_body(x_ref, w_ref, ha_ref, hb_ref, oa_ref, ob_ref):
  dis = _dis_of(ha_ref[...], hb_ref[...])
  xw = jnp.dot(x_ref[...], w_ref[...], preferred_element_type=jnp.float32)
  y1 = xw * dis
  oa_ref[...] = y1[:, :32]
  ob_ref[...] = y1[:, 32:]


def _k2_body(sa, sb, y1a, y1b, ha, hb, w2, b1r, oa, ob):
  dis = _dis_of(ha[...], hb[...])
  pre = jnp.concatenate([sa[0] + y1a[...], sb[0] + y1b[...]], axis=1)
  h = jnp.maximum(pre * dis + b1r[...], 0.0)
  y2 = jnp.dot(h, w2[...], preferred_element_type=jnp.float32) * dis
  oa[...] = y2[:, :64]
  ob[...] = y2[:, 64:]


def _k3_body(sa, sb, y2a, y2b, ha, hb, b2r, batchr, fcw, fcb, o, sums, cnts):
  g = pl.program_id(0)

  @pl.when(g == 0)
  def _():
    sums[...] = jnp.zeros_like(sums)
    cnts[...] = jnp.zeros_like(cnts)

  dis = _dis_of(ha[...], hb[...])
  h_lo = jnp.maximum((sa[0] + y2a[...]) * dis + b2r[:, :64], 0.0)
  h_hi = jnp.maximum((sb[0] + y2b[...]) * dis + b2r[:, 64:], 0.0)
  h = jnp.concatenate([h_lo, h_hi], axis=1)
  gid = lax.broadcasted_iota(jnp.int32, (_BM, _NG), 1)
  p = (batchr[...] == gid).astype(jnp.float32)
  dn = (((0,), (0,)), ((), ()))
  sums[...] += lax.dot_general(p, h, dn, preferred_element_type=jnp.float32)
  cnts[...] += lax.dot_general(p, jnp.ones_like(h), dn,
                               preferred_element_type=jnp.float32)

  @pl.when(g == _NBLK - 1)
  def _():
    pooled = sums[...] / jnp.maximum(cnts[...], 1.0)
    o[...] = jnp.dot(pooled, fcw[...],
                     preferred_element_type=jnp.float32) + fcb[...]


def _row_spec(d):
  return pl.BlockSpec((_BM, d), lambda i: (i, 0))


def _full_spec(shape):
  return pl.BlockSpec(shape, lambda i: tuple(0 for _ in shape))


def _half_spec(d, h):
  return pl.BlockSpec((1, _BM, d), lambda i, _h=h: (_h, i, 0))


def kernel(x, edge_index, batch, W1, b1, W2, b2, fc_W, fc_b):
  src = edge_index[0].astype(jnp.int32)
  dst = edge_index[1].astype(jnp.int32)
  npad = _EPAD - _E
  src2d = jnp.concatenate([src, jnp.zeros((npad,), jnp.int32)]).reshape(
      _NCHROWS * 2, 64)
  dst2d = jnp.concatenate([dst, jnp.full((npad,), _N, jnp.int32)]).reshape(
      _NCHROWS, _C)

  hist = _hist(dst2d)

  y1a, y1b = pl.pallas_call(
      _k1_body,
      grid=(_NBLK,),
      in_specs=[_row_spec(128), _full_spec((128, 64)),
                _half_spec(16, 0), _half_spec(16, 1)],
      out_specs=[_row_spec(32), _row_spec(32)],
      out_shape=[jax.ShapeDtypeStruct((_N, 32), jnp.float32),
                 jax.ShapeDtypeStruct((_N, 32), jnp.float32)],
  )(x, W1, hist, hist)

  s1 = _scat1(y1a, y1b, src2d, dst2d)

  y2a, y2b = pl.pallas_call(
      _k2_body,
      grid=(_NBLK,),
      in_specs=[_half_spec(32, 0), _half_spec(32, 1),
                _row_spec(32), _row_spec(32),
                _half_spec(16, 0), _half_spec(16, 1),
                _full_spec((64, 128)), _full_spec((1, 64))],
      out_specs=[_row_spec(64), _row_spec(64)],
      out_shape=[jax.ShapeDtypeStruct((_N, 64), jnp.float32),
                 jax.ShapeDtypeStruct((_N, 64), jnp.float32)],
  )(s1, s1, y1a, y1b, hist, hist, W2, b1.reshape(1, 64))

  s2 = _scat2(y2a, y2b, src2d, dst2d)

  out = pl.pallas_call(
      _k3_body,
      grid=(_NBLK,),
      in_specs=[_half_spec(64, 0), _half_spec(64, 1),
                _row_spec(64), _row_spec(64),
                _half_spec(16, 0), _half_spec(16, 1),
                _full_spec((1, 128)), pl.BlockSpec((_BM, 1), lambda i: (i, 0)),
                _full_spec((128, 2)), _full_spec((1, 2))],
      out_specs=_full_spec((_NG, 2)),
      out_shape=jax.ShapeDtypeStruct((_NG, 2), jnp.float32),
      scratch_shapes=[pltpu.VMEM((_NG, 128), jnp.float32),
                      pltpu.VMEM((_NG, 128), jnp.float32)],
  )(s2, s2, y2a, y2b, hist, hist, b2.reshape(1, 128),
    batch.astype(jnp.int32).reshape(_N, 1), fc_W, fc_b.reshape(1, 2))

  return out


# speedup vs baseline: 1.2803x; 1.0383x over previous
import functools

import jax
import jax.numpy as jnp
from jax import lax
from jax.experimental import pallas as pl
from jax.experimental.pallas import tpu as pltpu
from jax.experimental.pallas import tpu_sc as plsc

_N = 10000
_E = 320000
_NG = 64
_NC = 2
_NS = 16
_NW = _NC * _NS
_C = 128
_EPAD = 327680
_NCHROWS = _EPAD // _C
_NCH1 = _NCHROWS // _NW
_NCH2 = _NCHROWS // _NS
_NPAD = 10240
_RPT = _NPAD // _NS
_SLAB = 128
_NB = 4

_mesh = plsc.VectorSubcoreMesh(core_axis_name="c", subcore_axis_name="s")
_sc_params = pltpu.CompilerParams(use_tc_tiling_on_sc=False)


def _zero_slab(slab, d):
  def zrow(r, carry):
    def zcol(j, c2):
      slab[r, pl.ds(j * 16, 16)] = jnp.zeros((16,), jnp.float32)
      return c2
    return lax.fori_loop(0, d // 16, zcol, carry)
  lax.fori_loop(0, _SLAB, zrow, 0)


def _zero_acc_slab(slab, acc, s, d):
  _zero_slab(slab, d)
  for i in range(_RPT // _SLAB):
    pltpu.sync_copy(slab, acc.at[pl.ds(s * _RPT + i * _SLAB, _SLAB)])


def _dump_acc_slab(slab, acc, out_hbm, c, s):
  for i in range(_RPT // _SLAB):
    pltpu.sync_copy(acc.at[pl.ds(s * _RPT + i * _SLAB, _SLAB)], slab)
    pltpu.sync_copy(slab, out_hbm.at[pl.ds(c * _NPAD + s * _RPT + i * _SLAB,
                                           _SLAB)])


def _gs_pipeline(y_hbm, sidx_v, didx_v, rows, acc, gsem, ssem, nch):
  nb = len(rows)
  half = nb // 2
  banks = [rows[:half], rows[half:]]
  ng = nch // half

  def fire_g(g, bank):
    for b in range(half):
      for h in range(2):
        pltpu.async_copy(y_hbm.at[sidx_v.at[(g * half + b) * 2 + h]],
                         bank[b].at[pl.ds(h * 64, 64)], gsem)

  def wait_g(g, bank):
    for b in range(half):
      for h in range(2):
        pltpu.make_async_copy(y_hbm.at[sidx_v.at[(g * half + b) * 2 + h]],
                              bank[b].at[pl.ds(h * 64, 64)], gsem).wait()

  def fire_s(g, bank):
    for b in range(half):
      pltpu.async_copy(bank[b], acc.at[didx_v.at[g * half + b]], ssem,
                       add=True)

  def wait_s(g, bank):
    for b in range(half):
      pltpu.make_async_copy(bank[b], acc.at[didx_v.at[g * half + b]],
                            ssem).wait()

  fire_g(0, banks[0])
  fire_g(1, banks[1])

  def body(p, carry):
    ga = 2 * p
    gb = 2 * p + 1
    wait_g(ga, banks[0])
    fire_s(ga, banks[0])
    wait_g(gb, banks[1])
    fire_s(gb, banks[1])
    wait_s(ga, banks[0])

    @pl.when(ga + 2 < ng)
    def _():
      fire_g(ga + 2, banks[0])

    wait_s(gb, banks[1])

    @pl.when(gb + 2 < ng)
    def _():
      fire_g(gb + 2, banks[1])

    return carry

  lax.fori_loop(0, ng // 2, body, 0)


def _make_scatter_half(dh):
  @functools.partial(
      pl.kernel,
      mesh=_mesh,
      out_type=jax.ShapeDtypeStruct((_NC * _NPAD, dh), jnp.float32),
      scratch_types=(
          [pltpu.VMEM((_NCH2 * 2, 64), jnp.int32),
           pltpu.VMEM((_NCH2, _C), jnp.int32)]
          + [pltpu.VMEM((_C, dh), jnp.float32)] * _NB
          + [pltpu.VMEM((_SLAB, dh), jnp.float32),
             pltpu.VMEM_SHARED((_NPAD, dh), jnp.float32),
             pltpu.SemaphoreType.DMA,
             pltpu.SemaphoreType.DMA]
      ),
      compiler_params=_sc_params,
  )
  def k(ya_hbm, yb_hbm, src_hbm, dst_hbm, out_hbm, sidx_v, didx_v,
        r0, r1, r2, r3, slab, acc, gsem, ssem):
    c = lax.axis_index("c")
    s = lax.axis_index("s")

    pltpu.sync_copy(src_hbm.at[pl.ds(s * _NCH2 * 2, _NCH2 * 2)], sidx_v)
    pltpu.sync_copy(dst_hbm.at[pl.ds(s * _NCH2, _NCH2)], didx_v)
    _zero_acc_slab(slab, acc, s, dh)
    plsc.subcore_barrier()

    rows = [r0, r1, r2, r3]

    @pl.when(c == 0)
    def _():
      _gs_pipeline(ya_hbm, sidx_v, didx_v, rows, acc, gsem, ssem, _NCH2)

    @pl.when(c == 1)
    def _():
      _gs_pipeline(yb_hbm, sidx_v, didx_v, rows, acc, gsem, ssem, _NCH2)

    plsc.subcore_barrier()
    _dump_acc_slab(slab, acc, out_hbm, c, s)

  return k


_scat1 = _make_scatter_half(32)
_scat2 = _make_scatter_half(64)


@functools.partial(
    pl.kernel,
    mesh=_mesh,
    out_type=jax.ShapeDtypeStruct((_NC * _NPAD, 16), jnp.float32),
    scratch_types=[
        pltpu.VMEM((_NCH1, _C), jnp.int32),
        pltpu.VMEM((_C, 16), jnp.float32),
        pltpu.VMEM((_SLAB, 16), jnp.float32),
        pltpu.VMEM_SHARED((_NPAD, 16), jnp.float32),
        pltpu.SemaphoreType.DMA,
    ],
    compiler_params=_sc_params,
)
def _hist(dst_hbm, out_hbm, didx_v, ones_v, slab, acc, ssem):
  c = lax.axis_index("c")
  s = lax.axis_index("s")
  wid = s * _NC + c

  def orow(r, carry):
    ones_v[r, pl.ds(0, 16)] = jnp.ones((16,), jnp.float32)
    return carry
  lax.fori_loop(0, _C, orow, 0)

  pltpu.sync_copy(dst_hbm.at[pl.ds(wid * _NCH1, _NCH1)], didx_v)
  _zero_acc_slab(slab, acc, s, 16)
  plsc.subcore_barrier()

  nb = 8

  def group(t, carry):
    j0 = t * nb
    for b in range(nb):
      pltpu.async_copy(ones_v, acc.at[didx_v.at[j0 + b]], ssem, add=True)
    for b in range(nb):
      pltpu.make_async_copy(ones_v, acc.at[didx_v.at[j0 + b]], ssem).wait()
    return carry

  lax.fori_loop(0, _NCH1 // nb, group, 0)
  plsc.subcore_barrier()
  _dump_acc_slab(slab, acc, out_hbm, c, s)


_BM = 1000
_NBLK = _N // _BM


def _dis_of(ha, hb):
  deg = ha[:, 0:1] + hb[:, 0:1] + 1.0
  return lax.rsqrt(deg)


def _k1_body(x_ref, w_ref, ha_ref, hb_ref, oa_ref, ob_ref):
  dis = _dis_of(ha_ref[...], hb_ref[...])
  xw = jnp.dot(x_ref[...], w_ref[...], preferred_element_type=jnp.float32)
  y1 = xw * dis
  oa_ref[...] = y1[:, :32]
  ob_ref[...] = y1[:, 32:]


def _k2_body(sa, sb, y1a, y1b, ha, hb, w2, b1r, oa, ob):
  dis = _dis_of(ha[...], hb[...])
  pre = jnp.concatenate([sa[...] + y1a[...], sb[...] + y1b[...]], axis=1)
  h = jnp.maximum(pre * dis + b1r[...], 0.0)
  y2 = jnp.dot(h, w2[...], preferred_element_type=jnp.float32) * dis
  oa[...] = y2[:, :64]
  ob[...] = y2[:, 64:]


def _k3_body(sa, sb, y2a, y2b, ha, hb, b2r, batchr, fcw, fcb, o, sums, cnts):
  g = pl.program_id(0)

  @pl.when(g == 0)
  def _():
    sums[...] = jnp.zeros_like(sums)
    cnts[...] = jnp.zeros_like(cnts)

  dis = _dis_of(ha[...], hb[...])
  h_lo = jnp.maximum((sa[...] + y2a[...]) * dis + b2r[:, :64], 0.0)
  h_hi = jnp.maximum((sb[...] + y2b[...]) * dis + b2r[:, 64:], 0.0)
  h = jnp.concatenate([h_lo, h_hi], axis=1)
  gid = lax.broadcasted_iota(jnp.int32, (_BM, _NG), 1)
  p = (batchr[...] == gid).astype(jnp.float32)
  dn = (((0,), (0,)), ((), ()))
  sums[...] += lax.dot_general(p, h, dn, preferred_element_type=jnp.float32)
  cnts[...] += lax.dot_general(p, jnp.ones_like(h), dn,
                               preferred_element_type=jnp.float32)

  @pl.when(g == _NBLK - 1)
  def _():
    pooled = sums[...] / jnp.maximum(cnts[...], 1.0)
    o[...] = jnp.dot(pooled, fcw[...],
                     preferred_element_type=jnp.float32) + fcb[...]


def _row_spec(d):
  return pl.BlockSpec((_BM, d), lambda i: (i, 0))


def _full_spec(shape):
  return pl.BlockSpec(shape, lambda i: tuple(0 for _ in shape))


def kernel(x, edge_index, batch, W1, b1, W2, b2, fc_W, fc_b):
  src = edge_index[0].astype(jnp.int32)
  dst = edge_index[1].astype(jnp.int32)
  npad = _EPAD - _E
  src2d = jnp.concatenate([src, jnp.zeros((npad,), jnp.int32)]).reshape(
      _NCHROWS * 2, 64)
  dst2d = jnp.concatenate([dst, jnp.full((npad,), _N, jnp.int32)]).reshape(
      _NCHROWS, _C)

  hist = _hist(dst2d)
  ha, hb = hist[:_N], hist[_NPAD:_NPAD + _N]

  y1a, y1b = pl.pallas_call(
      _k1_body,
      grid=(_NBLK,),
      in_specs=[_row_spec(128), _full_spec((128, 64)),
                _row_spec(16), _row_spec(16)],
      out_specs=[_row_spec(32), _row_spec(32)],
      out_shape=[jax.ShapeDtypeStruct((_N, 32), jnp.float32),
                 jax.ShapeDtypeStruct((_N, 32), jnp.float32)],
  )(x, W1, ha, hb)

  s1 = _scat1(y1a, y1b, src2d, dst2d)

  y2a, y2b = pl.pallas_call(
      _k2_body,
      grid=(_NBLK,),
      in_specs=[_row_spec(32), _row_spec(32), _row_spec(32), _row_spec(32),
                _row_spec(16), _row_spec(16),
                _full_spec((64, 128)), _full_spec((1, 64))],
      out_specs=[_row_spec(64), _row_spec(64)],
      out_shape=[jax.ShapeDtypeStruct((_N, 64), jnp.float32),
                 jax.ShapeDtypeStruct((_N, 64), jnp.float32)],
  )(s1[:_N], s1[_NPAD:_NPAD + _N], y1a, y1b, ha, hb, W2, b1.reshape(1, 64))

  s2 = _scat2(y2a, y2b, src2d, dst2d)

  out = pl.pallas_call(
      _k3_body,
      grid=(_NBLK,),
      in_specs=[_row_spec(64), _row_spec(64), _row_spec(64), _row_spec(64),
                _row_spec(16), _row_spec(16),
                _full_spec((1, 128)), pl.BlockSpec((_BM, 1), lambda i: (i, 0)),
                _full_spec((128, 2)), _full_spec((1, 2))],
      out_specs=_full_spec((_NG, 2)),
      out_shape=jax.ShapeDtypeStruct((_NG, 2), jnp.float32),
      scratch_shapes=[pltpu.VMEM((_NG, 128), jnp.float32),
                      pltpu.VMEM((_NG, 128), jnp.float32)],
  )(s2[:_N], s2[_NPAD:_NPAD + _N], y2a, y2b, ha, hb, b2.reshape(1, 128),
    batch.astype(jnp.int32).reshape(_N, 1), fc_W, fc_b.reshape(1, 2))

  return out
